# baseline (speedup 1.0000x reference)
import functools

import jax
import jax.numpy as jnp
from jax import lax
from jax.experimental import pallas as pl
from jax.experimental.pallas import tpu as pltpu
from jax.experimental.pallas import tpu_sc as plsc

V = 1000
C = 1000
N = 51200
NC, NS, L = 2, 16, 16
NW = NC * NS
BPW = N // NW
RB = 8
NBLK = C // RB
S = 800
NSEG = N // S
UNITS = NBLK * NSEG
UPW = UNITS // NW
LSE_PAD = 1024


def _lse_body(table_ref, lse_ref):
    t = table_ref[...]
    m = jnp.max(t, axis=1)
    s = jnp.sum(jnp.exp(t - m[:, None]), axis=1)
    vals = jnp.log(s) + m
    lse_ref[...] = jnp.concatenate(
        [vals, jnp.zeros((LSE_PAD - V,), jnp.float32)])


def _loss_body(part_ref, out_ref):
    out_ref[...] = (jnp.sum(part_ref[...]) / N).reshape(1, 1)


def _sc_body(xf_hbm, tf_hbm, tt_hbm, ttf_hbm, lse_hbm, out_hbm, part_hbm,
             xa, ta, fidx_v, picked_v, lse_v, rows_v, out0, out1, acc_v,
             psem, osem0, osem1):
    wid = lax.axis_index("s") * NC + lax.axis_index("c")
    iota = lax.broadcasted_iota(jnp.int32, (L,), 0)

    pltpu.sync_copy(xf_hbm, xa)
    pltpu.sync_copy(tf_hbm.at[pl.ds(wid * BPW, BPW)], ta)
    pltpu.sync_copy(lse_hbm, lse_v)

    def fidx_step(g, _):
        xi = xa[pl.ds(wid * BPW + g * L, L)]
        ti = ta[pl.ds(g * L, L)]
        fidx_v[pl.ds(g * L, L)] = xi * C + ti
        return 0

    lax.fori_loop(0, BPW // L, fidx_step, 0)
    for j in range(BPW // 128):
        pltpu.async_copy(ttf_hbm.at[fidx_v.at[pl.ds(j * 128, 128)]],
                         picked_v.at[pl.ds(j * 128, 128)], psem)
    for j in range(BPW // 128):
        pltpu.make_async_copy(ttf_hbm.at[fidx_v.at[pl.ds(j * 128, 128)]],
                              picked_v.at[pl.ds(j * 128, 128)], psem).wait()

    def loss_step(g, acc):
        xi = xa[pl.ds(wid * BPW + g * L, L)]
        lses = plsc.load_gather(lse_v, [xi])
        return acc + (lses - picked_v[pl.ds(g * L, L)])

    acc = lax.fori_loop(0, BPW // L, loss_step, jnp.zeros((L,), jnp.float32))
    acc_v[...] = acc
    pltpu.sync_copy(acc_v, part_hbm.at[wid])

    u0 = wid * UPW
    obufs = ((out0, osem0), (out1, osem1))

    def step(i, _):
        for b in range(2):
            u = u0 + 2 * i + b
            ob, osem = obufs[b]
            tr = u // NSEG
            sg = u % NSEG
            i0 = sg * S

            @pl.when(jnp.logical_or(u == u0, sg == 0))
            def _():
                pltpu.sync_copy(tt_hbm.at[pl.ds(tr * RB, RB)], rows_v)

            @pl.when(i > 0)
            def _():
                pltpu.make_async_copy(
                    ob, out_hbm.at[pl.ds(0, RB), pl.ds(0, S)], osem).wait()

            def seg_step(g, _, ob=ob, i0=i0):
                xi = xa[pl.ds(i0 + g * L, L)]
                for r in range(RB):
                    rr = jnp.full((L,), r, jnp.int32)
                    vals = plsc.load_gather(rows_v, [rr, xi])
                    ob[r, pl.ds(g * L, L)] = vals
                return 0

            lax.fori_loop(0, S // L, seg_step, 0)
            pltpu.async_copy(
                ob, out_hbm.at[pl.ds(tr * RB, RB), pl.ds(i0, S)], osem)
        return 0

    lax.fori_loop(0, UPW // 2, step, 0)
    for b in range(2):
        ob, osem = obufs[b]
        pltpu.make_async_copy(
            ob, out_hbm.at[pl.ds(0, RB), pl.ds(0, S)], osem).wait()


_sc_gather = pl.kernel(
    _sc_body,
    out_type=(
        jax.ShapeDtypeStruct((C, N), jnp.float32),
        jax.ShapeDtypeStruct((NW, L), jnp.float32),
    ),
    mesh=plsc.VectorSubcoreMesh(core_axis_name="c", subcore_axis_name="s"),
    compiler_params=pltpu.CompilerParams(
        use_tc_tiling_on_sc=False, needs_layout_passes=False),
    scratch_types=[
        pltpu.VMEM((N,), jnp.int32),
        pltpu.VMEM((BPW,), jnp.int32),
        pltpu.VMEM((BPW,), jnp.int32),
        pltpu.VMEM((BPW,), jnp.float32),
        pltpu.VMEM((LSE_PAD,), jnp.float32),
        pltpu.VMEM((RB, C), jnp.float32),
        pltpu.VMEM((RB, S), jnp.float32),
        pltpu.VMEM((RB, S), jnp.float32),
        pltpu.VMEM((L,), jnp.float32),
        pltpu.SemaphoreType.DMA,
        pltpu.SemaphoreType.DMA,
        pltpu.SemaphoreType.DMA,
    ],
)

_lse_call = pl.pallas_call(
    _lse_body,
    out_shape=jax.ShapeDtypeStruct((LSE_PAD,), jnp.float32),
)

_loss_call = pl.pallas_call(
    _loss_body,
    out_shape=jax.ShapeDtypeStruct((1, 1), jnp.float32),
)


def kernel(x, targets, table):
    xf = x.reshape(-1).astype(jnp.int32)
    tf = targets.reshape(-1).astype(jnp.int32)
    table = table.astype(jnp.float32)
    tableT = jnp.swapaxes(table, 0, 1)
    lse = _lse_call(table)
    outT, partials = _sc_gather(xf, tf, tableT, table.reshape(-1), lse)
    loss = _loss_call(partials)[0, 0]
    return (jnp.swapaxes(outT, 0, 1), loss)


# speedup vs baseline: 2.0826x; 2.0826x over previous
import functools

import jax
import jax.numpy as jnp
from jax import lax
from jax.experimental import pallas as pl
from jax.experimental.pallas import tpu as pltpu
from jax.experimental.pallas import tpu_sc as plsc

V = 1000
C = 1000
N = 51200
NC, NS, L = 2, 16, 16
NW = NC * NS
BPW = N // NW
RB = 8
NBLK = C // RB
S = 800
NSEG = N // S
UNITS = NBLK * NSEG
UPW = UNITS // NW
LSE_PAD = 1024


def _lse_body(table_ref, lse_ref):
    t = table_ref[...]
    m = jnp.max(t, axis=1)
    s = jnp.sum(jnp.exp(t - m[:, None]), axis=1)
    vals = jnp.log(s) + m
    lse_ref[...] = jnp.concatenate(
        [vals, jnp.zeros((LSE_PAD - V,), jnp.float32)])


def _loss_body(part_ref, out_ref):
    out_ref[...] = (jnp.sum(part_ref[...]) / N).reshape(1, 1)


def _sc_body(xf_hbm, tf_hbm, tt_hbm, ttf_hbm, lse_hbm, out_hbm, part_hbm,
             xa, ta, fidx_v, picked_v, lse_v, rows_v, out0, out1, acc_v,
             psem, osem0, osem1):
    wid = lax.axis_index("s") * NC + lax.axis_index("c")
    iota = lax.broadcasted_iota(jnp.int32, (L,), 0)

    pltpu.sync_copy(xf_hbm, xa)
    pltpu.sync_copy(tf_hbm.at[pl.ds(wid * BPW, BPW)], ta)
    pltpu.sync_copy(lse_hbm, lse_v)

    def fidx_step(g, _):
        xi = xa[pl.ds(wid * BPW + g * L, L)]
        ti = ta[pl.ds(g * L, L)]
        fidx_v[pl.ds(g * L, L)] = xi * C + ti
        return 0

    lax.fori_loop(0, BPW // L, fidx_step, 0)
    for j in range(BPW // 128):
        pltpu.async_copy(ttf_hbm.at[fidx_v.at[pl.ds(j * 128, 128)]],
                         picked_v.at[pl.ds(j * 128, 128)], psem)
    for j in range(BPW // 128):
        pltpu.make_async_copy(ttf_hbm.at[fidx_v.at[pl.ds(j * 128, 128)]],
                              picked_v.at[pl.ds(j * 128, 128)], psem).wait()

    def loss_step(g, acc):
        xi = xa[pl.ds(wid * BPW + g * L, L)]
        lses = plsc.load_gather(lse_v, [xi])
        return acc + (lses - picked_v[pl.ds(g * L, L)])

    acc = lax.fori_loop(0, BPW // L, loss_step, jnp.zeros((L,), jnp.float32))
    acc_v[...] = acc
    pltpu.sync_copy(acc_v, part_hbm.at[wid])

    u0 = wid * UPW
    obufs = ((out0, osem0), (out1, osem1))

    def step(i, _):
        for b in range(2):
            u = u0 + 2 * i + b
            ob, osem = obufs[b]
            tr = u // NSEG
            sg = u % NSEG
            i0 = sg * S

            @pl.when(jnp.logical_or(u == u0, sg == 0))
            def _():
                pltpu.sync_copy(tt_hbm.at[pl.ds(tr * RB, RB)], rows_v)

            @pl.when(i > 0)
            def _():
                pltpu.make_async_copy(
                    ob, out_hbm.at[pl.ds(0, RB), pl.ds(0, S)], osem).wait()

            @plsc.parallel_loop(0, S // L, step=1, unroll=4)
            def _(g, ob=ob, i0=i0):
                xi = xa[pl.ds(i0 + g * L, L)]
                for r in range(RB):
                    rr = jnp.full((L,), r, jnp.int32)
                    vals = plsc.load_gather(rows_v, [rr, xi])
                    ob[r, pl.ds(g * L, L)] = vals
            pltpu.async_copy(
                ob, out_hbm.at[pl.ds(tr * RB, RB), pl.ds(i0, S)], osem)
        return 0

    lax.fori_loop(0, UPW // 2, step, 0)
    for b in range(2):
        ob, osem = obufs[b]
        pltpu.make_async_copy(
            ob, out_hbm.at[pl.ds(0, RB), pl.ds(0, S)], osem).wait()


_sc_gather = pl.kernel(
    _sc_body,
    out_type=(
        jax.ShapeDtypeStruct((C, N), jnp.float32),
        jax.ShapeDtypeStruct((NW, L), jnp.float32),
    ),
    mesh=plsc.VectorSubcoreMesh(core_axis_name="c", subcore_axis_name="s"),
    compiler_params=pltpu.CompilerParams(
        use_tc_tiling_on_sc=False, needs_layout_passes=False),
    scratch_types=[
        pltpu.VMEM((N,), jnp.int32),
        pltpu.VMEM((BPW,), jnp.int32),
        pltpu.VMEM((BPW,), jnp.int32),
        pltpu.VMEM((BPW,), jnp.float32),
        pltpu.VMEM((LSE_PAD,), jnp.float32),
        pltpu.VMEM((RB, C), jnp.float32),
        pltpu.VMEM((RB, S), jnp.float32),
        pltpu.VMEM((RB, S), jnp.float32),
        pltpu.VMEM((L,), jnp.float32),
        pltpu.SemaphoreType.DMA,
        pltpu.SemaphoreType.DMA,
        pltpu.SemaphoreType.DMA,
    ],
)

_lse_call = pl.pallas_call(
    _lse_body,
    out_shape=jax.ShapeDtypeStruct((LSE_PAD,), jnp.float32),
)

_loss_call = pl.pallas_call(
    _loss_body,
    out_shape=jax.ShapeDtypeStruct((1, 1), jnp.float32),
)


def kernel(x, targets, table):
    xf = x.reshape(-1).astype(jnp.int32)
    tf = targets.reshape(-1).astype(jnp.int32)
    table = table.astype(jnp.float32)
    tableT = jnp.swapaxes(table, 0, 1)
    lse = _lse_call(table)
    outT, partials = _sc_gather(xf, tf, tableT, table.reshape(-1), lse)
    loss = _loss_call(partials)[0, 0]
    return (jnp.swapaxes(outT, 0, 1), loss)


# speedup vs baseline: 2.4533x; 1.1780x over previous
import functools

import jax
import jax.numpy as jnp
from jax import lax
from jax.experimental import pallas as pl
from jax.experimental.pallas import tpu as pltpu
from jax.experimental.pallas import tpu_sc as plsc

V = 1000
C = 1000
N = 51200
NC, NS, L = 2, 16, 16
NW = NC * NS
BPW = N // NW
RB = 8
NBLK = C // RB
TL = 128
NTC = N // TL
S = 1024
TPS = S // TL
NSEG = N // S
UNITS = NBLK * NSEG
LSE_PAD = 1024


def _lse_body(table_ref, lse_ref):
    t = table_ref[...]
    m = jnp.max(t, axis=1)
    s = jnp.sum(jnp.exp(t - m[:, None]), axis=1)
    vals = jnp.log(s) + m
    lse_ref[...] = jnp.concatenate(
        [vals, jnp.zeros((LSE_PAD - V,), jnp.float32)])


def _loss_body(part_ref, out_ref):
    out_ref[...] = (jnp.sum(part_ref[...]) / N).reshape(1, 1)


def _sc_body(xf_hbm, tf_hbm, tt_hbm, tflat_hbm, lse_hbm, out_hbm, part_hbm,
             xa, ta, fidx_v, picked_v, lse_v, rows_v, out0, out1, acc_v,
             psem, osem0, osem1):
    wid = lax.axis_index("s") * NC + lax.axis_index("c")

    pltpu.sync_copy(xf_hbm, xa)
    pltpu.sync_copy(tf_hbm.at[pl.ds(wid * BPW, BPW)], ta)
    pltpu.sync_copy(lse_hbm, lse_v)

    def fidx_step(g, _):
        xi = xa[pl.ds(wid * BPW + g * L, L)]
        ti = ta[pl.ds(g * L, L)]
        fidx_v[pl.ds(g * L, L)] = xi * C + ti
        return 0

    lax.fori_loop(0, BPW // L, fidx_step, 0)
    for j in range(BPW // 128):
        pltpu.async_copy(tflat_hbm.at[fidx_v.at[pl.ds(j * 128, 128)]],
                         picked_v.at[pl.ds(j * 128, 128)], psem)
    for j in range(BPW // 128):
        pltpu.make_async_copy(tflat_hbm.at[fidx_v.at[pl.ds(j * 128, 128)]],
                              picked_v.at[pl.ds(j * 128, 128)], psem).wait()

    def loss_step(g, acc):
        xi = xa[pl.ds(wid * BPW + g * L, L)]
        lses = plsc.load_gather(lse_v, [xi])
        return acc + (lses - picked_v[pl.ds(g * L, L)])

    acc = lax.fori_loop(0, BPW // L, loss_step, jnp.zeros((L,), jnp.float32))
    acc_v[...] = acc
    pltpu.sync_copy(acc_v, part_hbm.at[wid])

    lo = (wid * UNITS) // NW
    hi = ((wid + 1) * UNITS) // NW
    npairs = (hi - lo) // 2
    obufs = ((out0, osem0), (out1, osem1))

    def do_unit(u, ob, first_for_worker):
        tr = u // NSEG
        sg = u % NSEG
        i0 = sg * S
        tc0 = sg * TPS

        @pl.when(jnp.logical_or(first_for_worker, sg == 0))
        def _():
            pltpu.sync_copy(tt_hbm.at[pl.ds(tr * RB, RB)], rows_v)

        @plsc.parallel_loop(0, TPS, step=1, unroll=2)
        def _(tc, ob=ob, i0=i0):
            for gg in range(TL // L):
                xi = xa[pl.ds(i0 + tc * TL + gg * L, L)]
                for r in range(RB):
                    rr = jnp.full((L,), r, jnp.int32)
                    vals = plsc.load_gather(rows_v, [rr, xi])
                    ob[tc, r, pl.ds(gg * L, L)] = vals
        return tr, tc0

    def pair_step(i, _):
        for b in range(2):
            u = lo + 2 * i + b
            ob, osem = obufs[b]

            @pl.when(i > 0)
            def _():
                pltpu.make_async_copy(
                    ob, out_hbm.at[0, pl.ds(0, TPS)], osem).wait()

            tr, tc0 = do_unit(u, ob, jnp.logical_and(i == 0, b == 0))
            pltpu.async_copy(ob, out_hbm.at[tr, pl.ds(tc0, TPS)], osem)
        return 0

    lax.fori_loop(0, npairs, pair_step, 0)

    @pl.when(hi - lo > 2 * npairs)
    def _():
        ob, osem = obufs[0]

        @pl.when(npairs > 0)
        def _():
            pltpu.make_async_copy(
                ob, out_hbm.at[0, pl.ds(0, TPS)], osem).wait()

        u = lo + 2 * npairs
        tr, tc0 = do_unit(u, ob, npairs == 0)
        pltpu.async_copy(ob, out_hbm.at[tr, pl.ds(tc0, TPS)], osem)

    pltpu.make_async_copy(out0, out_hbm.at[0, pl.ds(0, TPS)], osem0).wait()

    @pl.when(npairs > 0)
    def _():
        pltpu.make_async_copy(out1, out_hbm.at[0, pl.ds(0, TPS)], osem1).wait()


_sc_gather = pl.kernel(
    _sc_body,
    out_type=(
        jax.ShapeDtypeStruct((NBLK, NTC, RB, TL), jnp.float32),
        jax.ShapeDtypeStruct((NW, L), jnp.float32),
    ),
    mesh=plsc.VectorSubcoreMesh(core_axis_name="c", subcore_axis_name="s"),
    compiler_params=pltpu.CompilerParams(
        use_tc_tiling_on_sc=False, needs_layout_passes=False),
    scratch_types=[
        pltpu.VMEM((N,), jnp.int32),
        pltpu.VMEM((BPW,), jnp.int32),
        pltpu.VMEM((BPW,), jnp.int32),
        pltpu.VMEM((BPW,), jnp.float32),
        pltpu.VMEM((LSE_PAD,), jnp.float32),
        pltpu.VMEM((RB, C), jnp.float32),
        pltpu.VMEM((TPS, RB, TL), jnp.float32),
        pltpu.VMEM((TPS, RB, TL), jnp.float32),
        pltpu.VMEM((L,), jnp.float32),
        pltpu.SemaphoreType.DMA,
        pltpu.SemaphoreType.DMA,
        pltpu.SemaphoreType.DMA,
    ],
)

_lse_call = pl.pallas_call(
    _lse_body,
    out_shape=jax.ShapeDtypeStruct((LSE_PAD,), jnp.float32),
)

_loss_call = pl.pallas_call(
    _loss_body,
    out_shape=jax.ShapeDtypeStruct((1, 1), jnp.float32),
)


def kernel(x, targets, table):
    xf = x.reshape(-1).astype(jnp.int32)
    tf = targets.reshape(-1).astype(jnp.int32)
    table = table.astype(jnp.float32)
    tableT = jnp.swapaxes(table, 0, 1)
    lse = _lse_call(table)
    out4d, partials = _sc_gather(xf, tf, tableT, table.reshape(-1), lse)
    loss = _loss_call(partials)[0, 0]
    logits = (jnp.transpose(out4d, (1, 3, 0, 2))
              .reshape(N, C))
    return (logits, loss)
